# baseline (speedup 1.0000x reference)
import functools

import jax
import jax.numpy as jnp
from jax import lax
from jax.experimental import pallas as pl
from jax.experimental.pallas import tpu as pltpu
from jax.experimental.pallas import tpu_sc as plsc

B, N, M, NS, C, H = 1, 50000, 12500, 16, 128, 8
HC = C // H

NWORKERS = 32
GBATCH = 128
NBATCH = 50
RPT = NBATCH * GBATCH
R_PAD = NWORKERS * RPT
M_PAD = R_PAD // NS
BM = 256
GRID = M_PAD // BM


def _sc_gather_body(tab_hbm, idx_hbm, out_hbm, idx_v, buf, gsem):
    cc = lax.axis_index("c")
    ss = lax.axis_index("s")
    wid = ss * 2 + cc
    pltpu.sync_copy(idx_hbm.at[wid], idx_v)
    base0 = wid * NBATCH * GBATCH

    def body(j, carry):
        pltpu.async_copy(tab_hbm.at[idx_v.at[j]], buf, gsem).wait()
        pltpu.sync_copy(buf, out_hbm.at[pl.ds(base0 + j * GBATCH, GBATCH)])
        return carry

    lax.fori_loop(0, NBATCH, body, 0)


def _sc_gather(tab, idx3, width, tc_tiling):
    mesh = plsc.VectorSubcoreMesh(core_axis_name="c", subcore_axis_name="s")
    fn = functools.partial(
        pl.kernel,
        mesh=mesh,
        out_type=jax.ShapeDtypeStruct((R_PAD, width), jnp.float32),
        scratch_types=[
            pltpu.VMEM((NBATCH, GBATCH), jnp.int32),
            pltpu.VMEM((GBATCH, width), jnp.float32),
            pltpu.SemaphoreType.DMA,
        ],
        compiler_params=pltpu.CompilerParams(use_tc_tiling_on_sc=tc_tiling),
    )(_sc_gather_body)
    return fn(tab, idx3)


def _ln(x, g, b):
    mu = jnp.mean(x, axis=-1, keepdims=True)
    xc = x - mu
    var = jnp.mean(xc * xc, axis=-1, keepdims=True)
    return xc * lax.rsqrt(var + 1e-5) * g + b


def _tc_body(gf_ref, gp_ref, f2_ref, p2_ref, wq_ref, bq_ref, wk_ref, bk_ref,
             wv_ref, bv_ref, wpos_ref, bpos_ref, g1_ref, bn1_ref, wo_ref,
             bo_ref, g2_ref, bn2_ref, s1_ref, s2_ref, ones_ref, out_ref):
    f2 = f2_ref[...]
    q = jnp.dot(f2, wq_ref[...], preferred_element_type=jnp.float32) + bq_ref[...]
    gf = gf_ref[...]
    kf = jnp.dot(gf, wk_ref[...], preferred_element_type=jnp.float32) + bk_ref[...]
    vf = jnp.dot(gf, wv_ref[...], preferred_element_type=jnp.float32) + bv_ref[...]

    gp = gp_ref[...].reshape(BM, NS, 16)
    rel = p2_ref[...][:, None, :] - gp
    rel2 = (rel * rel).reshape(BM * NS, 16)
    disb = jnp.sqrt(jnp.dot(rel2, ones_ref[...],
                            preferred_element_type=jnp.float32))
    ef = jnp.maximum(disb * wpos_ref[...] + bpos_ref[...], 0.0)

    kh = (kf + ef).reshape(BM, NS, C)
    vh = vf + ef

    prod = q[:, None, :] * kh
    sc = jnp.dot(prod.reshape(BM * NS, C), s1_ref[...],
                 preferred_element_type=jnp.float32).reshape(BM, NS, H) * 0.25
    mx = jnp.max(sc, axis=1, keepdims=True)
    e = jnp.exp(sc - mx)
    e128 = jnp.dot(e.reshape(BM * NS, H), s2_ref[...],
                   preferred_element_type=jnp.float32)
    agg_un = jnp.sum((e128 * vh).reshape(BM, NS, C), axis=1)
    denom = jnp.dot(jnp.sum(e, axis=1), s2_ref[...],
                    preferred_element_type=jnp.float32)
    agg = agg_un / denom

    feature = _ln(f2 + agg, g1_ref[...], bn1_ref[...])
    y = jnp.dot(feature, wo_ref[...], preferred_element_type=jnp.float32) + bo_ref[...]
    out_ref[...] = _ln(y, g2_ref[...], bn2_ref[...])


def kernel(points1, points2, features1, features2, grouped_idx, W_pos, b_pos,
           Wq, bq, Wk, bk, Wv, bv, g1, bn1, Wo, bo, g2, bn2):
    f1 = features1[0]
    pts_tab = jnp.pad(points1[0], ((0, 0), (0, 13)))
    p2p = jnp.pad(points2[0], ((0, M_PAD - M), (0, 13)))
    f2p = jnp.pad(features2[0], ((0, M_PAD - M), (0, 0)))
    idx = grouped_idx[0].astype(jnp.int32).reshape(-1)
    idx3 = jnp.pad(idx, (0, R_PAD - M * NS)).reshape(NWORKERS, NBATCH, GBATCH)

    gf = _sc_gather(f1, idx3, C, True)
    gp = _sc_gather(pts_tab, idx3, 16, False)

    row2 = lambda a: a.reshape(1, C)
    s1 = (jnp.arange(C)[:, None] // HC == jnp.arange(H)[None, :]).astype(jnp.float32)
    s2 = s1.T

    full = lambda shape: pl.BlockSpec(shape, lambda i: (0, 0))
    out_tc = pl.pallas_call(
        _tc_body,
        grid=(GRID,),
        in_specs=[
            pl.BlockSpec((BM * NS, C), lambda i: (i, 0)),
            pl.BlockSpec((BM * NS, 16), lambda i: (i, 0)),
            pl.BlockSpec((BM, C), lambda i: (i, 0)),
            pl.BlockSpec((BM, 16), lambda i: (i, 0)),
            full((C, C)), full((1, C)),
            full((C, C)), full((1, C)),
            full((C, C)), full((1, C)),
            full((1, C)), full((1, C)),
            full((1, C)), full((1, C)),
            full((C, C)), full((1, C)),
            full((1, C)), full((1, C)),
            full((C, H)), full((H, C)), full((16, C)),
        ],
        out_specs=pl.BlockSpec((BM, C), lambda i: (i, 0)),
        out_shape=jax.ShapeDtypeStruct((M_PAD, C), jnp.float32),
    )(gf, gp, f2p, p2p, Wq, row2(bq), Wk, row2(bk), Wv, row2(bv),
      W_pos, row2(b_pos), row2(g1), row2(bn1), Wo, row2(bo), row2(g2),
      row2(bn2), s1, s2, jnp.ones((16, C), jnp.float32))

    return out_tc[:M][None]


# speedup vs baseline: 1.6228x; 1.6228x over previous
import functools

import jax
import jax.numpy as jnp
from jax import lax
from jax.experimental import pallas as pl
from jax.experimental.pallas import tpu as pltpu
from jax.experimental.pallas import tpu_sc as plsc

B, N, M, NS, C, H = 1, 50000, 12500, 16, 128, 8
HC = C // H

NWORKERS = 32
GBATCH = 128
NBATCH = 50
RPT = NBATCH * GBATCH
R_PAD = NWORKERS * RPT
M_PAD = R_PAD // NS
BM = 256
GRID = M_PAD // BM


def _sc_gather_body(tab_hbm, idx_hbm, out_hbm, idx_v, buf, gsem):
    cc = lax.axis_index("c")
    ss = lax.axis_index("s")
    wid = ss * 2 + cc
    pltpu.sync_copy(idx_hbm.at[wid], idx_v)
    base0 = wid * NBATCH * GBATCH

    def body(j, carry):
        pltpu.async_copy(tab_hbm.at[idx_v.at[j]], buf, gsem).wait()
        pltpu.sync_copy(buf, out_hbm.at[pl.ds(base0 + j * GBATCH, GBATCH)])
        return carry

    lax.fori_loop(0, NBATCH, body, 0)


def _sc_gather(tab, idx3, width, tc_tiling):
    mesh = plsc.VectorSubcoreMesh(core_axis_name="c", subcore_axis_name="s")
    fn = functools.partial(
        pl.kernel,
        mesh=mesh,
        out_type=jax.ShapeDtypeStruct((R_PAD, width), jnp.float32),
        scratch_types=[
            pltpu.VMEM((NBATCH, GBATCH), jnp.int32),
            pltpu.VMEM((GBATCH, width), jnp.float32),
            pltpu.SemaphoreType.DMA,
        ],
        compiler_params=pltpu.CompilerParams(use_tc_tiling_on_sc=tc_tiling),
    )(_sc_gather_body)
    return fn(tab, idx3)


def _ln(x, g, b):
    mu = jnp.mean(x, axis=-1, keepdims=True)
    xc = x - mu
    var = jnp.mean(xc * xc, axis=-1, keepdims=True)
    return xc * lax.rsqrt(var + 1e-5) * g + b


def _tc_body(gf_ref, gp_ref, f2_ref, p2_ref, wq_ref, bq_ref, wk_ref, bk_ref,
             wv_ref, bv_ref, wpos_ref, bpos_ref, g1_ref, bn1_ref, wo_ref,
             bo_ref, g2_ref, bn2_ref, s1_ref, s2_ref, ones_ref, out_ref):
    f2 = f2_ref[...]
    q = jnp.dot(f2, wq_ref[...], preferred_element_type=jnp.float32) + bq_ref[...]
    gf = gf_ref[...]
    kf = jnp.dot(gf, wk_ref[...], preferred_element_type=jnp.float32) + bk_ref[...]
    vf = jnp.dot(gf, wv_ref[...], preferred_element_type=jnp.float32) + bv_ref[...]

    gp = gp_ref[...].reshape(BM, NS, 16)
    rel = p2_ref[...][:, None, :] - gp
    rel2 = (rel * rel).reshape(BM * NS, 16)
    disb = jnp.sqrt(jnp.dot(rel2, ones_ref[...],
                            preferred_element_type=jnp.float32))
    ef = jnp.maximum(disb * wpos_ref[...] + bpos_ref[...], 0.0)

    kh = (kf + ef).reshape(BM, NS, C)
    vh = vf + ef

    prod = q[:, None, :] * kh
    sc = jnp.dot(prod.reshape(BM * NS, C), s1_ref[...],
                 preferred_element_type=jnp.float32).reshape(BM, NS, H) * 0.25
    mx = jnp.max(sc, axis=1, keepdims=True)
    e = jnp.exp(sc - mx)
    e128 = jnp.dot(e.reshape(BM * NS, H), s2_ref[...],
                   preferred_element_type=jnp.float32)
    agg_un = jnp.sum((e128 * vh).reshape(BM, NS, C), axis=1)
    denom = jnp.dot(jnp.sum(e, axis=1), s2_ref[...],
                    preferred_element_type=jnp.float32)
    agg = agg_un / denom

    feature = _ln(f2 + agg, g1_ref[...], bn1_ref[...])
    y = jnp.dot(feature, wo_ref[...], preferred_element_type=jnp.float32) + bo_ref[...]
    out_ref[...] = _ln(y, g2_ref[...], bn2_ref[...])


def kernel(points1, points2, features1, features2, grouped_idx, W_pos, b_pos,
           Wq, bq, Wk, bk, Wv, bv, g1, bn1, Wo, bo, g2, bn2):
    f1 = features1[0]
    pts_tab = jnp.pad(points1[0], ((0, 0), (0, 13)))
    p2p = jnp.pad(points2[0], ((0, M_PAD - M), (0, 13)))
    f2p = jnp.pad(features2[0], ((0, M_PAD - M), (0, 0)))
    idx = grouped_idx[0].astype(jnp.int32).reshape(-1)
    pad_ids = jnp.arange(R_PAD - M * NS, dtype=jnp.int32) * 7 % N
    idx3 = jnp.concatenate([idx, pad_ids]).reshape(NWORKERS, NBATCH, GBATCH)

    gf = _sc_gather(f1, idx3, C, True)
    gp = _sc_gather(pts_tab, idx3, 16, False)

    row2 = lambda a: a.reshape(1, C)
    s1 = (jnp.arange(C)[:, None] // HC == jnp.arange(H)[None, :]).astype(jnp.float32)
    s2 = s1.T

    full = lambda shape: pl.BlockSpec(shape, lambda i: (0, 0))
    out_tc = pl.pallas_call(
        _tc_body,
        grid=(GRID,),
        in_specs=[
            pl.BlockSpec((BM * NS, C), lambda i: (i, 0)),
            pl.BlockSpec((BM * NS, 16), lambda i: (i, 0)),
            pl.BlockSpec((BM, C), lambda i: (i, 0)),
            pl.BlockSpec((BM, 16), lambda i: (i, 0)),
            full((C, C)), full((1, C)),
            full((C, C)), full((1, C)),
            full((C, C)), full((1, C)),
            full((1, C)), full((1, C)),
            full((1, C)), full((1, C)),
            full((C, C)), full((1, C)),
            full((1, C)), full((1, C)),
            full((C, H)), full((H, C)), full((16, C)),
        ],
        out_specs=pl.BlockSpec((BM, C), lambda i: (i, 0)),
        out_shape=jax.ShapeDtypeStruct((M_PAD, C), jnp.float32),
    )(gf, gp, f2p, p2p, Wq, row2(bq), Wk, row2(bk), Wv, row2(bv),
      W_pos, row2(b_pos), row2(g1), row2(bn1), Wo, row2(bo), row2(g2),
      row2(bn2), s1, s2, jnp.ones((16, C), jnp.float32))

    return out_tc[:M][None]


# speedup vs baseline: 1.6837x; 1.0375x over previous
import functools

import jax
import jax.numpy as jnp
from jax import lax
from jax.experimental import pallas as pl
from jax.experimental.pallas import tpu as pltpu
from jax.experimental.pallas import tpu_sc as plsc

B, N, M, NS, C, H = 1, 50000, 12500, 16, 128, 8
HC = C // H

NWORKERS = 32
GBATCH = 128
NBATCH = 50
RPT = NBATCH * GBATCH
R_PAD = NWORKERS * RPT
M_PAD = R_PAD // NS
BM = 256
GRID = M_PAD // BM


def _sc_gather_body(tab_hbm, idx_hbm, out_hbm, idx_v, buf0, buf1, gs0, gs1):
    cc = lax.axis_index("c")
    ss = lax.axis_index("s")
    wid = ss * 2 + cc
    pltpu.sync_copy(idx_hbm.at[wid], idx_v)
    base0 = wid * NBATCH * GBATCH
    bufs = (buf0, buf1)
    gsems = (gs0, gs1)

    def gdesc(j, b):
        return pltpu.make_async_copy(tab_hbm.at[idx_v.at[j]], bufs[b], gsems[b])

    gdesc(0, 0).start()

    def body(g, carry):
        for b in range(2):
            j = 2 * g + b
            gdesc(j, b).wait()
            jn = jnp.minimum(j + 1, NBATCH - 1)
            gdesc(jn, 1 - b).start()
            pltpu.sync_copy(bufs[b],
                            out_hbm.at[pl.ds(base0 + j * GBATCH, GBATCH)])
        return carry

    lax.fori_loop(0, NBATCH // 2, body, 0)
    gdesc(NBATCH - 1, 0).wait()


def _sc_gather(tab, idx3, width, tc_tiling):
    mesh = plsc.VectorSubcoreMesh(core_axis_name="c", subcore_axis_name="s")
    fn = functools.partial(
        pl.kernel,
        mesh=mesh,
        out_type=jax.ShapeDtypeStruct((R_PAD, width), jnp.float32),
        scratch_types=[
            pltpu.VMEM((NBATCH, GBATCH), jnp.int32),
            pltpu.VMEM((GBATCH, width), jnp.float32),
            pltpu.VMEM((GBATCH, width), jnp.float32),
            pltpu.SemaphoreType.DMA,
            pltpu.SemaphoreType.DMA,
        ],
        compiler_params=pltpu.CompilerParams(use_tc_tiling_on_sc=tc_tiling),
    )(_sc_gather_body)
    return fn(tab, idx3)


def _ln(x, g, b):
    mu = jnp.mean(x, axis=-1, keepdims=True)
    xc = x - mu
    var = jnp.mean(xc * xc, axis=-1, keepdims=True)
    return xc * lax.rsqrt(var + 1e-5) * g + b


def _tc_body(gf_ref, gp_ref, f2_ref, p2_ref, wq_ref, bq_ref, wk_ref, bk_ref,
             wv_ref, bv_ref, wpos_ref, bpos_ref, g1_ref, bn1_ref, wo_ref,
             bo_ref, g2_ref, bn2_ref, s1_ref, s2_ref, ones_ref, out_ref):
    f2 = f2_ref[...]
    q = jnp.dot(f2, wq_ref[...], preferred_element_type=jnp.float32) + bq_ref[...]
    gf = gf_ref[...]
    kf = jnp.dot(gf, wk_ref[...], preferred_element_type=jnp.float32) + bk_ref[...]
    vf = jnp.dot(gf, wv_ref[...], preferred_element_type=jnp.float32) + bv_ref[...]

    gp = gp_ref[...].reshape(BM, NS, 16)
    rel = p2_ref[...][:, None, :] - gp
    rel2 = (rel * rel).reshape(BM * NS, 16)
    disb = jnp.sqrt(jnp.dot(rel2, ones_ref[...],
                            preferred_element_type=jnp.float32))
    ef = jnp.maximum(disb * wpos_ref[...] + bpos_ref[...], 0.0)

    kh = (kf + ef).reshape(BM, NS, C)
    vh = vf + ef

    prod = q[:, None, :] * kh
    sc = jnp.dot(prod.reshape(BM * NS, C), s1_ref[...],
                 preferred_element_type=jnp.float32).reshape(BM, NS, H) * 0.25
    mx = jnp.max(sc, axis=1, keepdims=True)
    e = jnp.exp(sc - mx)
    e128 = jnp.dot(e.reshape(BM * NS, H), s2_ref[...],
                   preferred_element_type=jnp.float32)
    agg_un = jnp.sum((e128 * vh).reshape(BM, NS, C), axis=1)
    denom = jnp.dot(jnp.sum(e, axis=1), s2_ref[...],
                    preferred_element_type=jnp.float32)
    agg = agg_un / denom

    feature = _ln(f2 + agg, g1_ref[...], bn1_ref[...])
    y = jnp.dot(feature, wo_ref[...], preferred_element_type=jnp.float32) + bo_ref[...]
    out_ref[...] = _ln(y, g2_ref[...], bn2_ref[...])


def kernel(points1, points2, features1, features2, grouped_idx, W_pos, b_pos,
           Wq, bq, Wk, bk, Wv, bv, g1, bn1, Wo, bo, g2, bn2):
    f1 = features1[0]
    pts_tab = jnp.pad(points1[0], ((0, 0), (0, 13)))
    p2p = jnp.pad(points2[0], ((0, M_PAD - M), (0, 13)))
    f2p = jnp.pad(features2[0], ((0, M_PAD - M), (0, 0)))
    idx = grouped_idx[0].astype(jnp.int32).reshape(-1)
    pad_ids = jnp.arange(R_PAD - M * NS, dtype=jnp.int32) * 7 % N
    idx3 = jnp.concatenate([idx, pad_ids]).reshape(NWORKERS, NBATCH, GBATCH)

    gf = _sc_gather(f1, idx3, C, True)
    gp = _sc_gather(pts_tab, idx3, 16, False)

    row2 = lambda a: a.reshape(1, C)
    s1 = (jnp.arange(C)[:, None] // HC == jnp.arange(H)[None, :]).astype(jnp.float32)
    s2 = s1.T

    full = lambda shape: pl.BlockSpec(shape, lambda i: (0, 0))
    out_tc = pl.pallas_call(
        _tc_body,
        grid=(GRID,),
        in_specs=[
            pl.BlockSpec((BM * NS, C), lambda i: (i, 0)),
            pl.BlockSpec((BM * NS, 16), lambda i: (i, 0)),
            pl.BlockSpec((BM, C), lambda i: (i, 0)),
            pl.BlockSpec((BM, 16), lambda i: (i, 0)),
            full((C, C)), full((1, C)),
            full((C, C)), full((1, C)),
            full((C, C)), full((1, C)),
            full((1, C)), full((1, C)),
            full((1, C)), full((1, C)),
            full((C, C)), full((1, C)),
            full((1, C)), full((1, C)),
            full((C, H)), full((H, C)), full((16, C)),
        ],
        out_specs=pl.BlockSpec((BM, C), lambda i: (i, 0)),
        out_shape=jax.ShapeDtypeStruct((M_PAD, C), jnp.float32),
    )(gf, gp, f2p, p2p, Wq, row2(bq), Wk, row2(bk), Wv, row2(bv),
      W_pos, row2(b_pos), row2(g1), row2(bn1), Wo, row2(bo), row2(g2),
      row2(bn2), s1, s2, jnp.ones((16, C), jnp.float32))

    return out_tc[:M][None]


# speedup vs baseline: 1.7550x; 1.0423x over previous
import functools

import jax
import jax.numpy as jnp
from jax import lax
from jax.experimental import pallas as pl
from jax.experimental.pallas import tpu as pltpu
from jax.experimental.pallas import tpu_sc as plsc

B, N, M, NS, C, H = 1, 50000, 12500, 16, 128, 8
HC = C // H

NWORKERS = 32
GBATCH = 128
NBATCH = 50
RPT = NBATCH * GBATCH
R_PAD = NWORKERS * RPT
M_PAD = R_PAD // NS
BM = 256
GRID = M_PAD // BM


def _sc_gather_body(tab_hbm, idx_hbm, out_hbm, idx_v, buf0, buf1, gs0, gs1):
    cc = lax.axis_index("c")
    ss = lax.axis_index("s")
    wid = ss * 2 + cc
    pltpu.sync_copy(idx_hbm.at[wid], idx_v)
    base0 = wid * NBATCH * GBATCH
    bufs = (buf0, buf1)
    gsems = (gs0, gs1)

    def gdesc(j, b):
        return pltpu.make_async_copy(tab_hbm.at[idx_v.at[j]], bufs[b], gsems[b])

    gdesc(0, 0).start()

    def body(g, carry):
        for b in range(2):
            j = 2 * g + b
            gdesc(j, b).wait()
            jn = jnp.minimum(j + 1, NBATCH - 1)
            gdesc(jn, 1 - b).start()
            pltpu.sync_copy(bufs[b],
                            out_hbm.at[pl.ds(base0 + j * GBATCH, GBATCH)])
        return carry

    lax.fori_loop(0, NBATCH // 2, body, 0)
    gdesc(NBATCH - 1, 0).wait()


def _sc_gather(tab, idx3, width, tc_tiling):
    mesh = plsc.VectorSubcoreMesh(core_axis_name="c", subcore_axis_name="s")
    fn = functools.partial(
        pl.kernel,
        mesh=mesh,
        out_type=jax.ShapeDtypeStruct((R_PAD, width), jnp.float32),
        scratch_types=[
            pltpu.VMEM((NBATCH, GBATCH), jnp.int32),
            pltpu.VMEM((GBATCH, width), jnp.float32),
            pltpu.VMEM((GBATCH, width), jnp.float32),
            pltpu.SemaphoreType.DMA,
            pltpu.SemaphoreType.DMA,
        ],
        compiler_params=pltpu.CompilerParams(use_tc_tiling_on_sc=tc_tiling),
    )(_sc_gather_body)
    return fn(tab, idx3)


def _ln(x, g, b):
    mu = jnp.mean(x, axis=-1, keepdims=True)
    xc = x - mu
    var = jnp.mean(xc * xc, axis=-1, keepdims=True)
    return xc * lax.rsqrt(var + 1e-5) * g + b


def _tc_body(gf_ref, gp_ref, f2_ref, p2_ref, wq_ref, bq_ref, wk_ref, bk_ref,
             wv_ref, bv_ref, wpos_ref, bpos_ref, g1_ref, bn1_ref, wo_ref,
             bo_ref, g2_ref, bn2_ref, s1_ref, s2_ref, out_ref):
    f2 = f2_ref[...]
    q = jnp.dot(f2, wq_ref[...], preferred_element_type=jnp.float32) + bq_ref[...]
    gf = gf_ref[...]
    kf = jnp.dot(gf, wk_ref[...], preferred_element_type=jnp.float32) + bk_ref[...]
    vf = jnp.dot(gf, wv_ref[...], preferred_element_type=jnp.float32) + bv_ref[...]

    gp = gp_ref[...].reshape(BM, NS, 16)
    rel = p2_ref[...][:, None, :] - gp
    dis = jnp.sqrt(jnp.sum(rel * rel, axis=-1))

    wpos = wpos_ref[...].reshape(1, 1, C)
    bpos = bpos_ref[...].reshape(1, 1, C)
    ef = jnp.maximum(dis[:, :, None] * wpos + bpos, 0.0)

    kh = kf.reshape(BM, NS, C) + ef
    vh = vf.reshape(BM, NS, C) + ef

    prod = q[:, None, :] * kh
    sc = jnp.dot(prod.reshape(BM * NS, C), s1_ref[...],
                 preferred_element_type=jnp.float32).reshape(BM, NS, H) * 0.25
    mx = jnp.max(sc, axis=1, keepdims=True)
    e = jnp.exp(sc - mx)
    attn = e / jnp.sum(e, axis=1, keepdims=True)
    a128 = jnp.dot(attn.reshape(BM * NS, H), s2_ref[...],
                   preferred_element_type=jnp.float32).reshape(BM, NS, C)
    agg = jnp.sum(a128 * vh, axis=1)

    feature = _ln(f2 + agg, g1_ref[...], bn1_ref[...])
    y = jnp.dot(feature, wo_ref[...], preferred_element_type=jnp.float32) + bo_ref[...]
    out_ref[...] = _ln(y, g2_ref[...], bn2_ref[...])


def kernel(points1, points2, features1, features2, grouped_idx, W_pos, b_pos,
           Wq, bq, Wk, bk, Wv, bv, g1, bn1, Wo, bo, g2, bn2):
    f1 = features1[0]
    pts_tab = jnp.pad(points1[0], ((0, 0), (0, 13)))
    p2p = jnp.pad(points2[0], ((0, M_PAD - M), (0, 13)))
    f2p = jnp.pad(features2[0], ((0, M_PAD - M), (0, 0)))
    idx = grouped_idx[0].astype(jnp.int32).reshape(-1)
    pad_ids = jnp.arange(R_PAD - M * NS, dtype=jnp.int32) * 7 % N
    idx3 = jnp.concatenate([idx, pad_ids]).reshape(NWORKERS, NBATCH, GBATCH)

    gf = _sc_gather(f1, idx3, C, True)
    gp = _sc_gather(pts_tab, idx3, 16, False)

    row2 = lambda a: a.reshape(1, C)
    s1 = (jnp.arange(C)[:, None] // HC == jnp.arange(H)[None, :]).astype(jnp.float32)
    s2 = s1.T

    full = lambda shape: pl.BlockSpec(shape, lambda i: (0, 0))
    out_tc = pl.pallas_call(
        _tc_body,
        grid=(GRID,),
        in_specs=[
            pl.BlockSpec((BM * NS, C), lambda i: (i, 0)),
            pl.BlockSpec((BM * NS, 16), lambda i: (i, 0)),
            pl.BlockSpec((BM, C), lambda i: (i, 0)),
            pl.BlockSpec((BM, 16), lambda i: (i, 0)),
            full((C, C)), full((1, C)),
            full((C, C)), full((1, C)),
            full((C, C)), full((1, C)),
            full((1, C)), full((1, C)),
            full((1, C)), full((1, C)),
            full((C, C)), full((1, C)),
            full((1, C)), full((1, C)),
            full((C, H)), full((H, C)),
        ],
        out_specs=pl.BlockSpec((BM, C), lambda i: (i, 0)),
        out_shape=jax.ShapeDtypeStruct((M_PAD, C), jnp.float32),
    )(gf, gp, f2p, p2p, Wq, row2(bq), Wk, row2(bk), Wv, row2(bv),
      W_pos, row2(b_pos), row2(g1), row2(bn1), Wo, row2(bo), row2(g2),
      row2(bn2), s1, s2)

    return out_tc[:M][None]


# speedup vs baseline: 1.8199x; 1.0369x over previous
import functools

import jax
import jax.numpy as jnp
from jax import lax
from jax.experimental import pallas as pl
from jax.experimental.pallas import tpu as pltpu
from jax.experimental.pallas import tpu_sc as plsc

B, N, M, NS, C, H = 1, 50000, 12500, 16, 128, 8
HC = C // H

NWORKERS = 32
GBATCH = 128
NCHUNK = 2
MC = M // NCHUNK
NBATCH = 26
RPT = NBATCH * GBATCH
R_PAD = NWORKERS * RPT
M_PAD = R_PAD // NS
BM = 256
GRID = M_PAD // BM


def _sc_gather_body(tab_hbm, idx_hbm, out_hbm, idx_v, buf0, buf1, gs0, gs1):
    cc = lax.axis_index("c")
    ss = lax.axis_index("s")
    wid = ss * 2 + cc
    pltpu.sync_copy(idx_hbm.at[wid], idx_v)
    base0 = wid * NBATCH * GBATCH
    bufs = (buf0, buf1)
    gsems = (gs0, gs1)

    def gdesc(j, b):
        return pltpu.make_async_copy(tab_hbm.at[idx_v.at[j]], bufs[b], gsems[b])

    gdesc(0, 0).start()

    def body(g, carry):
        for b in range(2):
            j = 2 * g + b
            gdesc(j, b).wait()
            jn = jnp.minimum(j + 1, NBATCH - 1)
            gdesc(jn, 1 - b).start()
            pltpu.sync_copy(bufs[b],
                            out_hbm.at[pl.ds(base0 + j * GBATCH, GBATCH)])
        return carry

    lax.fori_loop(0, NBATCH // 2, body, 0)
    gdesc(NBATCH - 1, 0).wait()


def _sc_gather(tab, idx3, width, tc_tiling):
    mesh = plsc.VectorSubcoreMesh(core_axis_name="c", subcore_axis_name="s")
    fn = functools.partial(
        pl.kernel,
        mesh=mesh,
        out_type=jax.ShapeDtypeStruct((R_PAD, width), jnp.float32),
        scratch_types=[
            pltpu.VMEM((NBATCH, GBATCH), jnp.int32),
            pltpu.VMEM((GBATCH, width), jnp.float32),
            pltpu.VMEM((GBATCH, width), jnp.float32),
            pltpu.SemaphoreType.DMA,
            pltpu.SemaphoreType.DMA,
        ],
        compiler_params=pltpu.CompilerParams(use_tc_tiling_on_sc=tc_tiling),
    )(_sc_gather_body)
    return fn(tab, idx3)


def _ln(x, g, b):
    mu = jnp.mean(x, axis=-1, keepdims=True)
    xc = x - mu
    var = jnp.mean(xc * xc, axis=-1, keepdims=True)
    return xc * lax.rsqrt(var + 1e-5) * g + b


def _tc_body(gf_ref, gp_ref, f2_ref, p2_ref, wq_ref, bq_ref, wk_ref, bk_ref,
             wv_ref, bv_ref, wpos_ref, bpos_ref, g1_ref, bn1_ref, wo_ref,
             bo_ref, g2_ref, bn2_ref, s1_ref, s2_ref, out_ref):
    f2 = f2_ref[...]
    q = jnp.dot(f2, wq_ref[...], preferred_element_type=jnp.float32) + bq_ref[...]
    gf = gf_ref[...]
    kf = jnp.dot(gf, wk_ref[...], preferred_element_type=jnp.float32) + bk_ref[...]
    vf = jnp.dot(gf, wv_ref[...], preferred_element_type=jnp.float32) + bv_ref[...]

    gp = gp_ref[...].reshape(BM, NS, 16)
    rel = p2_ref[...][:, None, :] - gp
    dis = jnp.sqrt(jnp.sum(rel * rel, axis=-1))

    wpos = wpos_ref[...].reshape(1, 1, C)
    bpos = bpos_ref[...].reshape(1, 1, C)
    ef = jnp.maximum(dis[:, :, None] * wpos + bpos, 0.0)

    kh = kf.reshape(BM, NS, C) + ef
    vh = vf.reshape(BM, NS, C) + ef

    prod = q[:, None, :] * kh
    sc = jnp.dot(prod.reshape(BM * NS, C), s1_ref[...],
                 preferred_element_type=jnp.float32).reshape(BM, NS, H) * 0.25
    mx = jnp.max(sc, axis=1, keepdims=True)
    e = jnp.exp(sc - mx)
    attn = e / jnp.sum(e, axis=1, keepdims=True)
    a128 = jnp.dot(attn.reshape(BM * NS, H), s2_ref[...],
                   preferred_element_type=jnp.float32).reshape(BM, NS, C)
    agg = jnp.sum(a128 * vh, axis=1)

    feature = _ln(f2 + agg, g1_ref[...], bn1_ref[...])
    y = jnp.dot(feature, wo_ref[...], preferred_element_type=jnp.float32) + bo_ref[...]
    out_ref[...] = _ln(y, g2_ref[...], bn2_ref[...])


def kernel(points1, points2, features1, features2, grouped_idx, W_pos, b_pos,
           Wq, bq, Wk, bk, Wv, bv, g1, bn1, Wo, bo, g2, bn2):
    f1 = features1[0]
    pts_tab = jnp.pad(points1[0], ((0, 0), (0, 13)))
    idx = grouped_idx[0].astype(jnp.int32).reshape(-1)
    pad_ids = jnp.arange(R_PAD - MC * NS, dtype=jnp.int32) * 7 % N

    row2 = lambda a: a.reshape(1, C)
    s1 = (jnp.arange(C)[:, None] // HC == jnp.arange(H)[None, :]).astype(jnp.float32)
    s2 = s1.T
    full = lambda shape: pl.BlockSpec(shape, lambda i: (0, 0))

    outs = []
    for c in range(NCHUNK):
        idx_c = lax.dynamic_slice(idx, (c * MC * NS,), (MC * NS,))
        idx3 = jnp.concatenate([idx_c, pad_ids]).reshape(NWORKERS, NBATCH,
                                                         GBATCH)
        gf = _sc_gather(f1, idx3, C, True)
        gp = _sc_gather(pts_tab, idx3, 16, False)
        p2p = jnp.pad(lax.dynamic_slice(points2[0], (c * MC, 0), (MC, 3)),
                      ((0, M_PAD - MC), (0, 13)))
        f2p = jnp.pad(lax.dynamic_slice(features2[0], (c * MC, 0), (MC, C)),
                      ((0, M_PAD - MC), (0, 0)))
        out_tc = pl.pallas_call(
            _tc_body,
            grid=(GRID,),
            in_specs=[
                pl.BlockSpec((BM * NS, C), lambda i: (i, 0)),
                pl.BlockSpec((BM * NS, 16), lambda i: (i, 0)),
                pl.BlockSpec((BM, C), lambda i: (i, 0)),
                pl.BlockSpec((BM, 16), lambda i: (i, 0)),
                full((C, C)), full((1, C)),
                full((C, C)), full((1, C)),
                full((C, C)), full((1, C)),
                full((1, C)), full((1, C)),
                full((1, C)), full((1, C)),
                full((C, C)), full((1, C)),
                full((1, C)), full((1, C)),
                full((C, H)), full((H, C)),
            ],
            out_specs=pl.BlockSpec((BM, C), lambda i: (i, 0)),
            out_shape=jax.ShapeDtypeStruct((M_PAD, C), jnp.float32),
        )(gf, gp, f2p, p2p, Wq, row2(bq), Wk, row2(bk), Wv, row2(bv),
          W_pos, row2(b_pos), row2(g1), row2(bn1), Wo, row2(bo), row2(g2),
          row2(bn2), s1, s2)
        outs.append(out_tc[:MC])

    return jnp.concatenate(outs)[None]
